# baseline (speedup 1.0000x reference)
import functools

import jax
import jax.numpy as jnp
from jax.experimental import pallas as pl

_BANDS = 2
_ROWS_PER_STEP = 16
_NCHUNKS = 8


def _band_rank(x, t, nchunks):
    r, n = x.shape
    cw = n // nchunks
    maxes = []
    for c in range(nchunks):
        xc = x[:, c * cw:(c + 1) * cw]
        colc = jax.lax.broadcasted_iota(jnp.int32, (r, cw), 1) + c * cw
        maxes.append(jnp.max(jnp.where(colc == t, xc, -jnp.inf),
                             axis=1, keepdims=True))
    vt = functools.reduce(jnp.maximum, maxes)
    cnts = []
    for c in range(nchunks):
        xc = x[:, c * cw:(c + 1) * cw]
        colc = jax.lax.broadcasted_iota(jnp.int32, (r, cw), 1) + c * cw
        pred = (xc > vt) | ((xc == vt) & (colc < t))
        cnts.append(jnp.sum(pred.astype(jnp.float32), axis=1, keepdims=True))
    return functools.reduce(jnp.add, cnts)


def _acc_kernel(*refs, scale, nchunks, bands, band_rows):
    x_refs = refs[:bands]
    t_ref = refs[bands]
    acc1_ref, acc5_ref = refs[bands + 1], refs[bands + 2]
    i = pl.program_id(0)
    r = x_refs[0].shape[1]
    a1 = jnp.zeros((1, 1), jnp.float32)
    a5 = jnp.zeros((1, 1), jnp.float32)
    for b in range(bands):
        x = x_refs[b][0]
        t = t_ref[pl.ds(b * band_rows + i * r, r), :]
        rank = _band_rank(x, t, nchunks)
        a1 = a1 + jnp.sum((rank < 1.0).astype(jnp.float32)).reshape(1, 1)
        a5 = a5 + jnp.sum((rank < 5.0).astype(jnp.float32)).reshape(1, 1)
    a1 = a1 * scale
    a5 = a5 * scale

    @pl.when(i == 0)
    def _init():
        acc1_ref[...] = a1
        acc5_ref[...] = a5

    @pl.when(i != 0)
    def _accum():
        acc1_ref[...] += a1
        acc5_ref[...] += a5


@jax.jit
def kernel(outputs, targets):
    b, n = outputs.shape
    bands = _BANDS
    band_rows = b // bands
    r = _ROWS_PER_STEP
    xr = outputs.reshape(bands, band_rows, n)
    t2 = targets.astype(jnp.int32).reshape(b, 1)
    body = functools.partial(_acc_kernel, scale=100.0 / b, nchunks=_NCHUNKS,
                             bands=bands, band_rows=band_rows)

    def make_spec(band):
        return pl.BlockSpec((1, r, n), lambda i, bb=band: (bb, i, 0))

    a1, a5 = pl.pallas_call(
        body,
        grid=(band_rows // r,),
        in_specs=[make_spec(band) for band in range(bands)] + [
            pl.BlockSpec((b, 1), lambda i: (0, 0)),
        ],
        out_specs=[
            pl.BlockSpec((1, 1), lambda i: (0, 0)),
            pl.BlockSpec((1, 1), lambda i: (0, 0)),
        ],
        out_shape=[
            jax.ShapeDtypeStruct((1, 1), jnp.float32),
            jax.ShapeDtypeStruct((1, 1), jnp.float32),
        ],
    )(*([xr] * bands + [t2]))
    return (a1.reshape(1), a5.reshape(1))


# speedup vs baseline: 1.0088x; 1.0088x over previous
import functools

import jax
import jax.numpy as jnp
from jax.experimental import pallas as pl

_BANDS = 4
_ROWS_PER_STEP = 8
_NCHUNKS = 4


def _band_rank(x, t, nchunks):
    r, n = x.shape
    cw = n // nchunks
    maxes = []
    for c in range(nchunks):
        xc = x[:, c * cw:(c + 1) * cw]
        colc = jax.lax.broadcasted_iota(jnp.int32, (r, cw), 1) + c * cw
        maxes.append(jnp.max(jnp.where(colc == t, xc, -jnp.inf),
                             axis=1, keepdims=True))
    vt = functools.reduce(jnp.maximum, maxes)
    cnts = []
    for c in range(nchunks):
        xc = x[:, c * cw:(c + 1) * cw]
        colc = jax.lax.broadcasted_iota(jnp.int32, (r, cw), 1) + c * cw
        pred = (xc > vt) | ((xc == vt) & (colc < t))
        cnts.append(jnp.sum(pred.astype(jnp.float32), axis=1, keepdims=True))
    return functools.reduce(jnp.add, cnts)


def _acc_kernel(*refs, scale, nchunks, bands, band_rows):
    x_refs = refs[:bands]
    t_ref = refs[bands]
    acc1_ref, acc5_ref = refs[bands + 1], refs[bands + 2]
    i = pl.program_id(0)
    r = x_refs[0].shape[1]
    a1 = jnp.zeros((1, 1), jnp.float32)
    a5 = jnp.zeros((1, 1), jnp.float32)
    for b in range(bands):
        x = x_refs[b][0]
        t = t_ref[pl.ds(b * band_rows + i * r, r), :]
        rank = _band_rank(x, t, nchunks)
        a1 = a1 + jnp.sum((rank < 1.0).astype(jnp.float32)).reshape(1, 1)
        a5 = a5 + jnp.sum((rank < 5.0).astype(jnp.float32)).reshape(1, 1)
    a1 = a1 * scale
    a5 = a5 * scale

    @pl.when(i == 0)
    def _init():
        acc1_ref[...] = a1
        acc5_ref[...] = a5

    @pl.when(i != 0)
    def _accum():
        acc1_ref[...] += a1
        acc5_ref[...] += a5


@jax.jit
def kernel(outputs, targets):
    b, n = outputs.shape
    bands = _BANDS
    band_rows = b // bands
    r = _ROWS_PER_STEP
    xr = outputs.reshape(bands, band_rows, n)
    t2 = targets.astype(jnp.int32).reshape(b, 1)
    body = functools.partial(_acc_kernel, scale=100.0 / b, nchunks=_NCHUNKS,
                             bands=bands, band_rows=band_rows)

    def make_spec(band):
        return pl.BlockSpec((1, r, n), lambda i, bb=band: (bb, i, 0))

    a1, a5 = pl.pallas_call(
        body,
        grid=(band_rows // r,),
        in_specs=[make_spec(band) for band in range(bands)] + [
            pl.BlockSpec((b, 1), lambda i: (0, 0)),
        ],
        out_specs=[
            pl.BlockSpec((1, 1), lambda i: (0, 0)),
            pl.BlockSpec((1, 1), lambda i: (0, 0)),
        ],
        out_shape=[
            jax.ShapeDtypeStruct((1, 1), jnp.float32),
            jax.ShapeDtypeStruct((1, 1), jnp.float32),
        ],
    )(*([xr] * bands + [t2]))
    return (a1.reshape(1), a5.reshape(1))
